# Initial kernel scaffold; baseline (speedup 1.0000x reference)
import jax
import jax.numpy as jnp
from jax.experimental import pallas as pl


def kernel(X, edge_index, batch, W1, b1, g1, be1, W2, b2, g2, be2, W3, b3, g3, be3, W4, b4, g4, be4, W5, b5, g5, be5, Wr, br):
    raise NotImplementedError("write your pallas kernel here")


# speedup vs baseline: 7.7084x; 7.7084x over previous
import functools

import jax
import jax.numpy as jnp
from jax import lax
from jax.experimental import pallas as pl
from jax.experimental.pallas import tpu as pltpu
from jax.experimental.pallas import tpu_sc as plsc

N = 10000
E = 320000
D = 128
G = 64

NC = 2
NS = 16
NW = NC * NS
CHUNK = 128
NCH = 79
E_PAD = NW * NCH * CHUNK
N_ACC = 10240
ROWS_PER_TILE = N_ACC // NS
YPAD = N + 16


def _sc_scatter_kernel(d, y_hbm, row_hbm, col_hbm, zero_hbm, out_hbm,
                       row_v, col_v, msg_v, acc_sh, sem):
    cid = lax.axis_index("c")
    sid = lax.axis_index("s")
    wid = sid * NC + cid

    pltpu.sync_copy(row_hbm.at[wid], row_v)
    pltpu.sync_copy(col_hbm.at[wid], col_v)

    pltpu.sync_copy(zero_hbm, acc_sh.at[pl.ds(sid * ROWS_PER_TILE, ROWS_PER_TILE)])
    plsc.subcore_barrier()

    def body(j, _):
        pltpu.async_copy(y_hbm.at[row_v.at[j]], msg_v, sem).wait()
        pltpu.sync_copy(msg_v, acc_sh.at[col_v.at[j]], add=True)
        return 0

    lax.fori_loop(0, NCH, body, 0)

    plsc.subcore_barrier()
    pltpu.sync_copy(acc_sh.at[pl.ds(sid * ROWS_PER_TILE, ROWS_PER_TILE)],
                    out_hbm.at[cid, pl.ds(sid * ROWS_PER_TILE, ROWS_PER_TILE)])


def _make_sc_scatter(d):
    mesh = plsc.VectorSubcoreMesh(core_axis_name="c", subcore_axis_name="s")
    return functools.partial(
        pl.kernel,
        out_type=jax.ShapeDtypeStruct((NC, N_ACC, d), jnp.float32),
        mesh=mesh,
        scratch_types=[
            pltpu.VMEM((NCH, CHUNK), jnp.int32),
            pltpu.VMEM((NCH, CHUNK), jnp.int32),
            pltpu.VMEM((CHUNK, d), jnp.float32),
            pltpu.VMEM_SHARED((N_ACC, d), jnp.float32),
            pltpu.SemaphoreType.DMA,
        ],
    )(functools.partial(_sc_scatter_kernel, d))


_sc_scatter_128 = _make_sc_scatter(D)


def _tc_pre_body(p_ref, x_ref, w_ref, dis_ref, y_ref):
    deg = p_ref[0, 0:N, 0:1] + p_ref[1, 0:N, 0:1] + 1.0
    dis = 1.0 / jnp.sqrt(deg)
    dis2d = jnp.broadcast_to(dis, (N, D))
    dis_ref[...] = dis2d
    xw = jnp.dot(x_ref[...], w_ref[...], preferred_element_type=jnp.float32,
                 precision=lax.Precision.HIGHEST)
    y_ref[0:N, :] = dis2d * xw
    y_ref[N:YPAD, :] = jnp.zeros((YPAD - N, D), jnp.float32)


def _tc_layer_body(has_res, sp_ref, y_ref, dis_ref, b_ref, g_ref, be_ref,
                   xprev_ref, wn_ref, x_ref, yn_ref):
    s = sp_ref[0, 0:N, :] + sp_ref[1, 0:N, :] + y_ref[0:N, :]
    conv = dis_ref[...] * s + b_ref[...]
    h = jnp.maximum(conv, 0.0)
    mean = jnp.mean(h, axis=0, keepdims=True)
    dlt = h - mean
    var = jnp.mean(dlt * dlt, axis=0, keepdims=True)
    bn = dlt / jnp.sqrt(var + 1e-5) * g_ref[...] + be_ref[...]
    if has_res:
        x = bn + xprev_ref[...]
    else:
        x = bn
    x_ref[...] = x
    xw = jnp.dot(x, wn_ref[...], preferred_element_type=jnp.float32,
                 precision=lax.Precision.HIGHEST)
    yn_ref[0:N, :] = dis_ref[...] * xw
    yn_ref[N:YPAD, :] = jnp.zeros((YPAD - N, D), jnp.float32)


def _tc_final_body(sp_ref, y_ref, dis_ref, b_ref, g_ref, be_ref, xprev_ref,
                   batch_ref, wr_ref, br_ref, pred_ref, xm_ref):
    s = sp_ref[0, 0:N, :] + sp_ref[1, 0:N, :] + y_ref[0:N, :]
    conv = dis_ref[...] * s + b_ref[...]
    h = jnp.maximum(conv, 0.0)
    mean = jnp.mean(h, axis=0, keepdims=True)
    dlt = h - mean
    var = jnp.mean(dlt * dlt, axis=0, keepdims=True)
    bn = dlt / jnp.sqrt(var + 1e-5) * g_ref[...] + be_ref[...]
    x5 = bn + xprev_ref[...]

    gids = lax.broadcasted_iota(jnp.int32, (N, G), 1)
    m = (batch_ref[...] == gids).astype(jnp.float32)
    ssum = lax.dot_general(m, x5, (((0,), (0,)), ((), ())),
                           preferred_element_type=jnp.float32,
                           precision=lax.Precision.HIGHEST)
    cnt = lax.dot_general(m, jnp.ones((N, D), jnp.float32),
                          (((0,), (0,)), ((), ())),
                          preferred_element_type=jnp.float32,
                          precision=lax.Precision.HIGHEST)
    xm = ssum / jnp.maximum(cnt, 1.0)
    xm_ref[...] = xm
    pred_ref[...] = jnp.dot(xm, wr_ref[...], preferred_element_type=jnp.float32,
                            precision=lax.Precision.HIGHEST) + br_ref[...]


_tc_pre = pl.pallas_call(
    _tc_pre_body,
    out_shape=(jax.ShapeDtypeStruct((N, D), jnp.float32),
               jax.ShapeDtypeStruct((YPAD, D), jnp.float32)),
)

_tc_layer_res = pl.pallas_call(
    functools.partial(_tc_layer_body, True),
    out_shape=(jax.ShapeDtypeStruct((N, D), jnp.float32),
               jax.ShapeDtypeStruct((YPAD, D), jnp.float32)),
)

_tc_layer_nores = pl.pallas_call(
    functools.partial(_tc_layer_body, False),
    out_shape=(jax.ShapeDtypeStruct((N, D), jnp.float32),
               jax.ShapeDtypeStruct((YPAD, D), jnp.float32)),
)

_tc_final = pl.pallas_call(
    _tc_final_body,
    out_shape=(jax.ShapeDtypeStruct((G, 1), jnp.float32),
               jax.ShapeDtypeStruct((G, D), jnp.float32)),
)


def kernel(X, edge_index, batch, W1, b1, g1, be1, W2, b2, g2, be2, W3, b3,
           g3, be3, W4, b4, g4, be4, W5, b5, g5, be5, Wr, br):
    f32 = jnp.float32
    row = edge_index[0].astype(jnp.int32)
    col = edge_index[1].astype(jnp.int32)
    pad = E_PAD - E
    row_p = jnp.concatenate([row, jnp.full((pad,), N, jnp.int32)]).reshape(NW, NCH, CHUNK)
    col_p = jnp.concatenate([col, jnp.zeros((pad,), jnp.int32)]).reshape(NW, NCH, CHUNK)

    zeros128 = jnp.zeros((ROWS_PER_TILE, D), f32)
    ones_tab = jnp.concatenate([jnp.ones((N, D), f32),
                                jnp.zeros((YPAD - N, D), f32)])

    degp = _sc_scatter_128(ones_tab, row_p, col_p, zeros128)

    b1r, g1r, be1r = b1.reshape(1, D), g1.reshape(1, D), be1.reshape(1, D)
    b2r, g2r, be2r = b2.reshape(1, D), g2.reshape(1, D), be2.reshape(1, D)
    b3r, g3r, be3r = b3.reshape(1, D), g3.reshape(1, D), be3.reshape(1, D)
    b4r, g4r, be4r = b4.reshape(1, D), g4.reshape(1, D), be4.reshape(1, D)
    b5r, g5r, be5r = b5.reshape(1, D), g5.reshape(1, D), be5.reshape(1, D)

    dis2d, y1 = _tc_pre(degp, X, W1)

    sp1 = _sc_scatter_128(y1, row_p, col_p, zeros128)
    x1, y2 = _tc_layer_nores(sp1, y1, dis2d, b1r, g1r, be1r, X, W2)

    sp2 = _sc_scatter_128(y2, row_p, col_p, zeros128)
    x2, y3 = _tc_layer_res(sp2, y2, dis2d, b2r, g2r, be2r, x1, W3)

    sp3 = _sc_scatter_128(y3, row_p, col_p, zeros128)
    x3, y4 = _tc_layer_res(sp3, y3, dis2d, b3r, g3r, be3r, x2, W4)

    sp4 = _sc_scatter_128(y4, row_p, col_p, zeros128)
    x4, y5 = _tc_layer_res(sp4, y4, dis2d, b4r, g4r, be4r, x3, W5)

    sp5 = _sc_scatter_128(y5, row_p, col_p, zeros128)
    pred, x_mean = _tc_final(sp5, y5, dis2d, b5r, g5r, be5r, x4,
                             batch.astype(jnp.int32).reshape(N, 1), Wr,
                             br.reshape(1, 1))
    return (pred, x_mean)
